# baseline (speedup 1.0000x reference)
import functools

import jax
import jax.numpy as jnp
from jax.experimental import pallas as pl
from jax.experimental.pallas import tpu as pltpu

N, H = 65536, 1024
BLK = 512


def _sigmoid(x):
    return 1.0 / (1.0 + jnp.exp2(x * -1.4426950408889634))


def _body(h_ref, tg_ref, x0_ref, t_ref, wsc_ref, w1h_ref, w1r_ref,
          w2_ref, b2_ref, w3p_ref, scal_ref, out_ref):
    bs = scal_ref[0]
    bc = scal_ref[1]
    wc_last = scal_ref[2]
    b3 = scal_ref[3]

    h = h_ref[...].astype(jnp.bfloat16)
    sc = jnp.dot(h, wsc_ref[...], preferred_element_type=jnp.float32)
    s_logits = sc[:, 0:1] + bs
    c_pre = sc[:, 1:2] + s_logits * wc_last + bc
    c_lambda = (jnp.maximum(c_pre, 0.0)
                + jnp.log1p(jnp.exp(-jnp.abs(c_pre))) + 1e-6)

    tg = tg_ref[...]
    x0 = x0_ref[...]
    tv = t_ref[...]
    mask = (tg > 0.0).astype(jnp.float32)
    y = jnp.log1p(jnp.maximum(tg, 0.0))
    x_t = (1.0 - tv) * x0 + tv * y
    target_v = y - x0

    e4 = jnp.concatenate(
        [c_lambda, x_t, tv, jnp.ones_like(tv)], axis=1).astype(jnp.bfloat16)
    z1p = (jnp.dot(h, w1h_ref[...], preferred_element_type=jnp.float32)
           + jnp.dot(e4, w1r_ref[...], preferred_element_type=jnp.float32))
    z1 = (z1p * _sigmoid(z1p)).astype(jnp.bfloat16)
    z2p = jnp.dot(z1, w2_ref[...], preferred_element_type=jnp.float32) + b2_ref[...]
    z2 = z2p * _sigmoid(z2p)
    pv = jnp.dot(z2, w3p_ref[...], preferred_element_type=jnp.float32)[:, 0:1] + b3

    r = pv - target_v
    main_v = mask * r * r
    bce_v = (jnp.maximum(s_logits, 0.0) - s_logits * mask
             + jnp.log1p(jnp.exp(-jnp.abs(s_logits))))
    c_d = c_lambda - y
    c_v = c_d * c_d

    lane = jax.lax.broadcasted_iota(jnp.int32, (h.shape[0], 128), 1)
    acc = (jnp.where(lane == 0, bce_v, 0.0)
           + jnp.where(lane == 1, c_v, 0.0)
           + jnp.where(lane == 2, main_v, 0.0)
           + jnp.where(lane == 3, mask, 0.0))
    out_ref[...] = jnp.sum(acc, axis=0, keepdims=True)[None]


@jax.jit
def kernel(h, targets, x0, t, Ws, bs, Wc, bc, W1, b1, W2, b2, W3, b3):
    f32 = jnp.float32
    bf16 = jnp.bfloat16
    wsc = (jnp.zeros((H, 128), f32).at[:, 0].set(Ws).at[:, 1].set(Wc[:H])
           .astype(bf16))
    w3p = jnp.zeros((H, 128), f32).at[:, 0].set(W3).astype(bf16)
    scal = jnp.stack([bs, bc, Wc[H], b3]).astype(f32)

    nblk = N // BLK
    parts = pl.pallas_call(
        _body,
        grid=(nblk,),
        in_specs=[
            pl.BlockSpec((BLK, H), lambda i: (i, 0)),
            pl.BlockSpec((BLK, 1), lambda i: (i, 0)),
            pl.BlockSpec((BLK, 1), lambda i: (i, 0)),
            pl.BlockSpec((BLK, 1), lambda i: (i, 0)),
            pl.BlockSpec((H, 128), lambda i: (0, 0)),
            pl.BlockSpec((H, H), lambda i: (0, 0)),
            pl.BlockSpec((4, H), lambda i: (0, 0)),
            pl.BlockSpec((H, H), lambda i: (0, 0)),
            pl.BlockSpec((1, H), lambda i: (0, 0)),
            pl.BlockSpec((H, 128), lambda i: (0, 0)),
            pl.BlockSpec(memory_space=pltpu.SMEM),
        ],
        out_specs=pl.BlockSpec((1, 1, 128), lambda i: (i, 0, 0)),
        out_shape=jax.ShapeDtypeStruct((nblk, 1, 128), f32),
        compiler_params=pltpu.CompilerParams(
            dimension_semantics=("arbitrary",),
            vmem_limit_bytes=100 * 1024 * 1024,
        ),
    )(h, targets[:, None], x0[:, None], t[:, None],
      wsc, W1[:H].astype(bf16),
      jnp.concatenate([W1[H:], b1[None, :]], axis=0).astype(bf16),
      W2.astype(bf16), b2[None, :], w3p, scal)

    sums = jnp.sum(parts[:, 0, :], axis=0)
    s_loss = sums[0] / N
    c_loss = sums[1] / N
    main_loss = sums[2] / jnp.maximum(sums[3], 1.0)
    total = main_loss + 0.05 * s_loss + 0.05 * c_loss
    return jnp.stack([main_loss, s_loss, c_loss, total])


# speedup vs baseline: 1.3004x; 1.3004x over previous
import jax
import jax.numpy as jnp
from jax.experimental import pallas as pl
from jax.experimental.pallas import tpu as pltpu

N, H = 65536, 1024
BLK = 512
SUB = 2


def _sigmoid(x):
    return 0.5 * jnp.tanh(x * 0.5) + 0.5


def _body(h_ref, tg_ref, x0_ref, t_ref,
          wss_ref, wsc_ref, w1h_ref, w1r_ref, w2_ref, b2_ref, w3d_ref,
          scal_ref, hout_ref, mout_ref, hb_scr, e_scr, tv_scr, mk_scr):
    i = pl.program_id(0)
    slot = jax.lax.rem(i, 2)
    prev = 1 - slot

    @pl.when(i == 0)
    def _init():
        e_scr[...] = jnp.zeros_like(e_scr)

    hrows = []
    mrows = []
    sb = BLK // SUB
    for g in range(SUB):
        rs = slice(g * sb, (g + 1) * sb)
        hrow, mrow = _half(
            h_ref.at[rs, :], tg_ref.at[rs, :], x0_ref.at[rs, :],
            t_ref.at[rs, :], wss_ref, wsc_ref, w1h_ref, w1r_ref, w2_ref,
            b2_ref, w3d_ref, scal_ref,
            hb_scr.at[slot, rs, :], e_scr.at[slot, rs, :],
            tv_scr.at[slot, rs, :], mk_scr.at[slot, rs, :],
            hb_scr.at[prev, rs, :], e_scr.at[prev, rs, :],
            tv_scr.at[prev, rs, :], mk_scr.at[prev, rs, :])
        hrows.append(hrow)
        mrows.append(mrow)
    hout_ref[...] = sum(hrows)[None]
    mout_ref[...] = sum(mrows)[None]


def _half(h_ref, tg_ref, x0_ref, t_ref,
          wss_ref, wsc_ref, w1h_ref, w1r_ref, w2_ref, b2_ref, w3d_ref,
          scal_ref, hb_wr, e_wr, tv_wr, mk_wr, hb_rd, e_rd, tv_rd, mk_rd):
    SB = BLK // SUB
    lane = jax.lax.broadcasted_iota(jnp.int32, (SB, 128), 1)

    b3 = scal_ref[3]
    e = e_rd[...].astype(jnp.bfloat16)
    hp = hb_rd[...]
    z1p = (jnp.dot(hp, w1h_ref[...], preferred_element_type=jnp.float32)
           + jnp.dot(e, w1r_ref[...], preferred_element_type=jnp.float32))
    z1 = (z1p * _sigmoid(z1p)).astype(jnp.bfloat16)
    z2p = jnp.dot(z1, w2_ref[...], preferred_element_type=jnp.float32) + b2_ref[...]
    z2 = (z2p * _sigmoid(z2p)).astype(jnp.bfloat16)
    pv = jnp.dot(z2, w3d_ref[...], preferred_element_type=jnp.float32) + b3

    maskp = mk_rd[...]
    r = pv - tv_rd[...]
    main_v = maskp * r * r
    macc = jnp.where(lane == 2, main_v, jnp.where(lane == 3, maskp, 0.0))
    mrow = jnp.sum(macc, axis=0, keepdims=True)

    bs = scal_ref[0]
    bc = scal_ref[1]
    wc_last = scal_ref[2]

    hb = h_ref[...].astype(jnp.bfloat16)
    hb_wr[...] = hb
    s_logits = jnp.dot(hb, wss_ref[...], preferred_element_type=jnp.float32) + bs
    c_pre = (jnp.dot(hb, wsc_ref[...], preferred_element_type=jnp.float32)
             + s_logits * wc_last + bc)
    c_lambda = (jnp.maximum(c_pre, 0.0)
                + jnp.log1p(jnp.exp(-jnp.abs(c_pre))) + 1e-6)

    tg = jnp.broadcast_to(tg_ref[...], (SB, 128))
    x0 = jnp.broadcast_to(x0_ref[...], (SB, 128))
    tv = jnp.broadcast_to(t_ref[...], (SB, 128))
    mask = (tg > 0.0).astype(jnp.float32)
    y = jnp.log1p(jnp.maximum(tg, 0.0))
    x_t = (1.0 - tv) * x0 + tv * y

    e_wr[...] = jnp.where(
        lane == 0, c_lambda,
        jnp.where(lane == 1, x_t,
                  jnp.where(lane == 2, tv,
                            jnp.where(lane == 3, 1.0, 0.0))))
    tv_wr[...] = y - x0
    mk_wr[...] = mask

    bce_v = (jnp.maximum(s_logits, 0.0) - s_logits * mask
             + jnp.log1p(jnp.exp(-jnp.abs(s_logits))))
    c_d = c_lambda - y
    c_v = c_d * c_d
    hacc = jnp.where(lane == 0, bce_v, jnp.where(lane == 1, c_v, 0.0))
    hrow = jnp.sum(hacc, axis=0, keepdims=True)
    return hrow, mrow


@jax.jit
def kernel(h, targets, x0, t, Ws, bs, Wc, bc, W1, b1, W2, b2, W3, b3):
    f32 = jnp.float32
    bf16 = jnp.bfloat16
    wss = jnp.broadcast_to(Ws[:, None], (H, 128)).astype(bf16)
    wsc = jnp.broadcast_to(Wc[:H, None], (H, 128)).astype(bf16)
    w3d = jnp.broadcast_to(W3[:, None], (H, 128)).astype(bf16)
    w1r = (jnp.zeros((128, H), f32)
           .at[0].set(W1[H]).at[1].set(W1[H + 1]).at[2].set(W1[H + 2])
           .at[3].set(b1).astype(bf16))
    scal = jnp.stack([bs, bc, Wc[H], b3]).astype(f32)

    nblk = N // BLK
    last = nblk - 1

    def blk_i(i):
        return (jnp.minimum(i, last), 0)

    head_parts, mlp_parts = pl.pallas_call(
        _body,
        grid=(nblk + 1,),
        in_specs=[
            pl.BlockSpec((BLK, H), blk_i),
            pl.BlockSpec((BLK, 1), blk_i),
            pl.BlockSpec((BLK, 1), blk_i),
            pl.BlockSpec((BLK, 1), blk_i),
            pl.BlockSpec((H, 128), lambda i: (0, 0)),
            pl.BlockSpec((H, 128), lambda i: (0, 0)),
            pl.BlockSpec((H, H), lambda i: (0, 0)),
            pl.BlockSpec((128, H), lambda i: (0, 0)),
            pl.BlockSpec((H, H), lambda i: (0, 0)),
            pl.BlockSpec((1, H), lambda i: (0, 0)),
            pl.BlockSpec((H, 128), lambda i: (0, 0)),
            pl.BlockSpec(memory_space=pltpu.SMEM),
        ],
        out_specs=[
            pl.BlockSpec((1, 1, 128), lambda i: (jnp.minimum(i, last), 0, 0)),
            pl.BlockSpec((1, 1, 128), lambda i: (jnp.maximum(i - 1, 0), 0, 0)),
        ],
        out_shape=[
            jax.ShapeDtypeStruct((nblk, 1, 128), f32),
            jax.ShapeDtypeStruct((nblk, 1, 128), f32),
        ],
        scratch_shapes=[
            pltpu.VMEM((2, BLK, H), bf16),
            pltpu.VMEM((2, BLK, 128), f32),
            pltpu.VMEM((2, BLK, 128), f32),
            pltpu.VMEM((2, BLK, 128), f32),
        ],
        compiler_params=pltpu.CompilerParams(
            dimension_semantics=("arbitrary",),
            vmem_limit_bytes=100 * 1024 * 1024,
        ),
    )(h, targets[:, None], x0[:, None], t[:, None],
      wss, wsc, W1[:H].astype(bf16), w1r, W2.astype(bf16), b2[None, :],
      w3d, scal)

    sums = jnp.sum(head_parts[:, 0, :], axis=0) + jnp.sum(mlp_parts[:, 0, :], axis=0)
    s_loss = sums[0] / N
    c_loss = sums[1] / N
    main_loss = sums[2] / jnp.maximum(sums[3], 1.0)
    total = main_loss + 0.05 * s_loss + 0.05 * c_loss
    return jnp.stack([main_loss, s_loss, c_loss, total])


# speedup vs baseline: 1.4669x; 1.1280x over previous
import jax
import jax.numpy as jnp
from jax.experimental import pallas as pl
from jax.experimental.pallas import tpu as pltpu

N, H = 65536, 1024
BLK = 512
SUB = 2


def _silu(x):
    return x * (0.5 * jnp.tanh(x * 0.5) + 0.5)


def _body(h_ref, tg_ref, x0_ref, t_ref,
          wss_ref, wsc_ref, w1h_ref, w1c_ref, w1e_ref, w2_ref, b2_ref,
          w3d_ref, scal_ref, out_ref):
    sb = BLK // SUB
    rows = []
    for g in range(SUB):
        rs = slice(g * sb, (g + 1) * sb)
        rows.append(_chain(
            h_ref.at[rs, :], tg_ref.at[rs, :], x0_ref.at[rs, :],
            t_ref.at[rs, :], wss_ref, wsc_ref, w1h_ref, w1c_ref, w1e_ref,
            w2_ref, b2_ref, w3d_ref, scal_ref))
    out_ref[...] = sum(rows)[None]


def _chain(h_ref, tg_ref, x0_ref, t_ref,
           wss_ref, wsc_ref, w1h_ref, w1c_ref, w1e_ref, w2_ref, b2_ref,
           w3d_ref, scal_ref):
    SB = BLK // SUB
    bs = scal_ref[0]
    bc = scal_ref[1]
    wc_last = scal_ref[2]
    b3 = scal_ref[3]

    lane = jax.lax.broadcasted_iota(jnp.int32, (SB, 128), 1)

    h = h_ref[...].astype(jnp.bfloat16)
    s_logits = jnp.dot(h, wss_ref[...], preferred_element_type=jnp.float32) + bs
    c_pre = (jnp.dot(h, wsc_ref[...], preferred_element_type=jnp.float32)
             + s_logits * wc_last + bc)
    c_lambda = (jnp.maximum(c_pre, 0.0)
                + jnp.log1p(jnp.exp(-jnp.abs(c_pre))) + 1e-6)

    tg = jnp.broadcast_to(tg_ref[...], (SB, 128))
    x0 = jnp.broadcast_to(x0_ref[...], (SB, 128))
    tv = jnp.broadcast_to(t_ref[...], (SB, 128))
    mask = (tg > 0.0).astype(jnp.float32)
    y = jnp.log1p(jnp.maximum(tg, 0.0))
    x_t = (1.0 - tv) * x0 + tv * y
    target_v = y - x0

    e3 = jnp.where(lane == 0, x_t,
                   jnp.where(lane == 1, tv,
                             jnp.where(lane == 2, 1.0, 0.0))).astype(jnp.bfloat16)
    z1p = (jnp.dot(h, w1h_ref[...], preferred_element_type=jnp.float32)
           + jnp.dot(e3, w1e_ref[...], preferred_element_type=jnp.float32)
           + pltpu.repeat(c_lambda, H // 128, axis=1) * w1c_ref[...])
    z1b = z1p.astype(jnp.bfloat16)
    z1 = z1b * (jnp.tanh(z1b * 0.5) * 0.5 + 0.5)
    z2p = jnp.dot(z1, w2_ref[...], preferred_element_type=jnp.float32) + b2_ref[...]
    z2b = z2p.astype(jnp.bfloat16)
    z2 = z2b * (jnp.tanh(z2b * 0.5) * 0.5 + 0.5)
    pv = jnp.dot(z2, w3d_ref[...], preferred_element_type=jnp.float32) + b3

    r = pv - target_v
    main_v = mask * r * r
    bce_v = (jnp.maximum(s_logits, 0.0) - s_logits * mask
             + jnp.log1p(jnp.exp(-jnp.abs(s_logits))))
    c_d = c_lambda - y
    c_v = c_d * c_d

    acc = jnp.where(
        lane == 0, bce_v,
        jnp.where(lane == 1, c_v,
                  jnp.where(lane == 2, main_v,
                            jnp.where(lane == 3, mask, 0.0))))
    return jnp.sum(acc, axis=0, keepdims=True)


@jax.jit
def kernel(h, targets, x0, t, Ws, bs, Wc, bc, W1, b1, W2, b2, W3, b3):
    f32 = jnp.float32
    bf16 = jnp.bfloat16
    wss = jnp.broadcast_to(Ws[:, None], (H, 128)).astype(bf16)
    wsc = jnp.broadcast_to(Wc[:H, None], (H, 128)).astype(bf16)
    w3d = jnp.broadcast_to(W3[:, None], (H, 128)).astype(bf16)
    w1c = W1[H][None, :]
    w1e = (jnp.zeros((128, H), f32)
           .at[0].set(W1[H + 1]).at[1].set(W1[H + 2]).at[2].set(b1)
           .astype(bf16))
    scal = jnp.stack([bs, bc, Wc[H], b3]).astype(f32)

    nblk = N // BLK
    parts = pl.pallas_call(
        _body,
        grid=(nblk,),
        in_specs=[
            pl.BlockSpec((BLK, H), lambda i: (i, 0)),
            pl.BlockSpec((BLK, 1), lambda i: (i, 0)),
            pl.BlockSpec((BLK, 1), lambda i: (i, 0)),
            pl.BlockSpec((BLK, 1), lambda i: (i, 0)),
            pl.BlockSpec((H, 128), lambda i: (0, 0)),
            pl.BlockSpec((H, 128), lambda i: (0, 0)),
            pl.BlockSpec((H, H), lambda i: (0, 0)),
            pl.BlockSpec((1, H), lambda i: (0, 0)),
            pl.BlockSpec((128, H), lambda i: (0, 0)),
            pl.BlockSpec((H, H), lambda i: (0, 0)),
            pl.BlockSpec((1, H), lambda i: (0, 0)),
            pl.BlockSpec((H, 128), lambda i: (0, 0)),
            pl.BlockSpec(memory_space=pltpu.SMEM),
        ],
        out_specs=pl.BlockSpec((1, 1, 128), lambda i: (i, 0, 0)),
        out_shape=jax.ShapeDtypeStruct((nblk, 1, 128), f32),
        compiler_params=pltpu.CompilerParams(
            dimension_semantics=("arbitrary",),
            vmem_limit_bytes=100 * 1024 * 1024,
        ),
    )(h, targets[:, None], x0[:, None], t[:, None],
      wss, wsc, W1[:H].astype(bf16), w1c, w1e, W2.astype(bf16),
      b2[None, :], w3d, scal)

    sums = jnp.sum(parts[:, 0, :], axis=0)
    s_loss = sums[0] / N
    c_loss = sums[1] / N
    main_loss = sums[2] / jnp.maximum(sums[3], 1.0)
    total = main_loss + 0.05 * s_loss + 0.05 * c_loss
    return jnp.stack([main_loss, s_loss, c_loss, total])


# speedup vs baseline: 1.5417x; 1.0510x over previous
import jax
import jax.numpy as jnp
from jax.experimental import pallas as pl
from jax.experimental.pallas import tpu as pltpu

N, H = 65536, 1024
BLK = 512


def _silu(x):
    return x * (0.5 * jnp.tanh(x * 0.5) + 0.5)


def _body(h_ref, tg_ref, x0_ref, t_ref, wsc_ref, w1h_ref, w1r_ref,
          w2_ref, b2_ref, w3p_ref, scal_ref, out_ref):
    bs = scal_ref[0]
    bc = scal_ref[1]
    wc_last = scal_ref[2]
    b3 = scal_ref[3]

    h = h_ref[...]
    sc = jnp.dot(h, wsc_ref[...], preferred_element_type=jnp.float32)
    s_logits = sc[:, 0:1] + bs
    c_pre = sc[:, 1:2] + s_logits * wc_last + bc
    c_lambda = (jnp.maximum(c_pre, 0.0)
                + jnp.log1p(jnp.exp(-jnp.abs(c_pre))) + 1e-6)

    tg = tg_ref[...]
    x0 = x0_ref[...]
    tv = t_ref[...]
    mask = (tg > 0.0).astype(jnp.float32)
    y = jnp.log1p(jnp.maximum(tg, 0.0))
    x_t = (1.0 - tv) * x0 + tv * y
    target_v = y - x0

    z1p = (jnp.dot(h, w1h_ref[...], preferred_element_type=jnp.float32)
           + c_lambda * w1r_ref[0:1, :]
           + x_t * w1r_ref[1:2, :]
           + tv * w1r_ref[2:3, :]
           + w1r_ref[3:4, :])
    z1 = _silu(z1p)
    z2p = jnp.dot(z1, w2_ref[...], preferred_element_type=jnp.float32) + b2_ref[...]
    z2 = _silu(z2p)
    pv = jnp.dot(z2, w3p_ref[...], preferred_element_type=jnp.float32)[:, 0:1] + b3

    r = pv - target_v
    main_v = mask * r * r
    bce_v = (jnp.maximum(s_logits, 0.0) - s_logits * mask
             + jnp.log1p(jnp.exp(-jnp.abs(s_logits))))
    c_d = c_lambda - y
    c_v = c_d * c_d

    lane = jax.lax.broadcasted_iota(jnp.int32, (BLK, 128), 1)
    acc = (jnp.where(lane == 0, bce_v, 0.0)
           + jnp.where(lane == 1, c_v, 0.0)
           + jnp.where(lane == 2, main_v, 0.0)
           + jnp.where(lane == 3, mask, 0.0))
    out_ref[...] = jnp.sum(acc, axis=0, keepdims=True)[None]


@jax.jit
def kernel(h, targets, x0, t, Ws, bs, Wc, bc, W1, b1, W2, b2, W3, b3):
    f32 = jnp.float32
    wsc = jnp.zeros((H, 128), f32).at[:, 0].set(Ws).at[:, 1].set(Wc[:H])
    w3p = jnp.zeros((H, 128), f32).at[:, 0].set(W3)
    scal = jnp.stack([bs, bc, Wc[H], b3]).astype(f32)

    nblk = N // BLK
    parts = pl.pallas_call(
        _body,
        grid=(nblk,),
        in_specs=[
            pl.BlockSpec((BLK, H), lambda i: (i, 0)),
            pl.BlockSpec((BLK, 1), lambda i: (i, 0)),
            pl.BlockSpec((BLK, 1), lambda i: (i, 0)),
            pl.BlockSpec((BLK, 1), lambda i: (i, 0)),
            pl.BlockSpec((H, 128), lambda i: (0, 0)),
            pl.BlockSpec((H, H), lambda i: (0, 0)),
            pl.BlockSpec((4, H), lambda i: (0, 0)),
            pl.BlockSpec((H, H), lambda i: (0, 0)),
            pl.BlockSpec((1, H), lambda i: (0, 0)),
            pl.BlockSpec((H, 128), lambda i: (0, 0)),
            pl.BlockSpec(memory_space=pltpu.SMEM),
        ],
        out_specs=pl.BlockSpec((1, 1, 128), lambda i: (i, 0, 0)),
        out_shape=jax.ShapeDtypeStruct((nblk, 1, 128), f32),
        compiler_params=pltpu.CompilerParams(
            dimension_semantics=("arbitrary",),
            vmem_limit_bytes=100 * 1024 * 1024,
        ),
    )(h, targets[:, None], x0[:, None], t[:, None],
      wsc, W1[:H],
      jnp.concatenate([W1[H:], b1[None, :]], axis=0),
      W2, b2[None, :], w3p, scal)

    sums = jnp.sum(parts[:, 0, :], axis=0)
    s_loss = sums[0] / N
    c_loss = sums[1] / N
    main_loss = sums[2] / jnp.maximum(sums[3], 1.0)
    total = main_loss + 0.05 * s_loss + 0.05 * c_loss
    return jnp.stack([main_loss, s_loss, c_loss, total])


# speedup vs baseline: 1.6765x; 1.0874x over previous
import jax
import jax.numpy as jnp
from jax.experimental import pallas as pl
from jax.experimental.pallas import tpu as pltpu

N, H = 65536, 1024
BLK = 1024


def _silu(x):
    return x * (0.5 * jnp.tanh(x * 0.5) + 0.5)


def _body(h_ref, tg_ref, x0_ref, t_ref, wsc_ref, w1h_ref, w1r_ref,
          w2_ref, b2_ref, w3p_ref, scal_ref, out_ref):
    bs = scal_ref[0]
    bc = scal_ref[1]
    wc_last = scal_ref[2]
    b3 = scal_ref[3]

    h = h_ref[...]
    sc = jnp.dot(h, wsc_ref[...], preferred_element_type=jnp.float32)
    s_logits = sc[:, 0:1] + bs
    c_pre = sc[:, 1:2] + s_logits * wc_last + bc
    c_lambda = (jnp.maximum(c_pre, 0.0)
                + jnp.log1p(jnp.exp(-jnp.abs(c_pre))) + 1e-6)

    tg = tg_ref[...]
    x0 = x0_ref[...]
    tv = t_ref[...]
    mask = (tg > 0.0).astype(jnp.float32)
    y = jnp.log1p(jnp.maximum(tg, 0.0))
    x_t = (1.0 - tv) * x0 + tv * y
    target_v = y - x0

    z1p = (jnp.dot(h, w1h_ref[...], preferred_element_type=jnp.float32)
           + c_lambda * w1r_ref[0:1, :]
           + x_t * w1r_ref[1:2, :]
           + tv * w1r_ref[2:3, :]
           + w1r_ref[3:4, :])
    z1 = _silu(z1p)
    z2p = jnp.dot(z1, w2_ref[...], preferred_element_type=jnp.float32) + b2_ref[...]
    z2 = _silu(z2p)
    pv = jnp.dot(z2, w3p_ref[...], preferred_element_type=jnp.float32)[:, 0:1] + b3

    r = pv - target_v
    main_v = mask * r * r
    bce_v = (jnp.maximum(s_logits, 0.0) - s_logits * mask
             + jnp.log1p(jnp.exp(-jnp.abs(s_logits))))
    c_d = c_lambda - y
    c_v = c_d * c_d

    lane = jax.lax.broadcasted_iota(jnp.int32, (BLK, 128), 1)
    acc = (jnp.where(lane == 0, bce_v, 0.0)
           + jnp.where(lane == 1, c_v, 0.0)
           + jnp.where(lane == 2, main_v, 0.0)
           + jnp.where(lane == 3, mask, 0.0))
    out_ref[...] = jnp.sum(acc, axis=0, keepdims=True)[None]


@jax.jit
def kernel(h, targets, x0, t, Ws, bs, Wc, bc, W1, b1, W2, b2, W3, b3):
    f32 = jnp.float32
    wsc = jnp.zeros((H, 128), f32).at[:, 0].set(Ws).at[:, 1].set(Wc[:H])
    w3p = jnp.zeros((H, 128), f32).at[:, 0].set(W3)
    scal = jnp.stack([bs, bc, Wc[H], b3]).astype(f32)

    nblk = N // BLK
    parts = pl.pallas_call(
        _body,
        grid=(nblk,),
        in_specs=[
            pl.BlockSpec((BLK, H), lambda i: (i, 0)),
            pl.BlockSpec((BLK, 1), lambda i: (i, 0)),
            pl.BlockSpec((BLK, 1), lambda i: (i, 0)),
            pl.BlockSpec((BLK, 1), lambda i: (i, 0)),
            pl.BlockSpec((H, 128), lambda i: (0, 0)),
            pl.BlockSpec((H, H), lambda i: (0, 0)),
            pl.BlockSpec((4, H), lambda i: (0, 0)),
            pl.BlockSpec((H, H), lambda i: (0, 0)),
            pl.BlockSpec((1, H), lambda i: (0, 0)),
            pl.BlockSpec((H, 128), lambda i: (0, 0)),
            pl.BlockSpec(memory_space=pltpu.SMEM),
        ],
        out_specs=pl.BlockSpec((1, 1, 128), lambda i: (i, 0, 0)),
        out_shape=jax.ShapeDtypeStruct((nblk, 1, 128), f32),
        compiler_params=pltpu.CompilerParams(
            dimension_semantics=("arbitrary",),
            vmem_limit_bytes=100 * 1024 * 1024,
        ),
    )(h, targets[:, None], x0[:, None], t[:, None],
      wsc, W1[:H],
      jnp.concatenate([W1[H:], b1[None, :]], axis=0),
      W2, b2[None, :], w3p, scal)

    sums = jnp.sum(parts[:, 0, :], axis=0)
    s_loss = sums[0] / N
    c_loss = sums[1] / N
    main_loss = sums[2] / jnp.maximum(sums[3], 1.0)
    total = main_loss + 0.05 * s_loss + 0.05 * c_loss
    return jnp.stack([main_loss, s_loss, c_loss, total])
